# baseline (speedup 1.0000x reference)
import functools

import jax
import jax.numpy as jnp
from jax import lax
from jax.experimental import pallas as pl
from jax.experimental.pallas import tpu as pltpu
from jax.experimental.pallas import tpu_sc as plsc

_NW = 32
_L = 16


def _sigmoid(x):
    return 1.0 / (1.0 + jnp.exp(-x))


def _stage_a(xs, src_pad, dst_pad, guards2, e_total, ini, fin):
    e_pad = src_pad.shape[0]
    q = ini.shape[0]
    nb, d = xs.shape
    chunk = e_pad // _NW
    n_steps = chunk // _L
    mesh = plsc.VectorSubcoreMesh(core_axis_name="c", subcore_axis_name="s")

    @functools.partial(
        pl.kernel,
        out_type=jax.ShapeDtypeStruct((_NW * nb * _L,), jnp.float32),
        mesh=mesh,
        compiler_params=pltpu.CompilerParams(needs_layout_passes=False),
        scratch_types=[
            pltpu.VMEM((chunk,), jnp.int32),
            pltpu.VMEM((chunk,), jnp.int32),
            pltpu.VMEM((q,), jnp.float32),
            pltpu.VMEM((q,), jnp.float32),
            pltpu.VMEM((chunk + _L,), jnp.float32),
            pltpu.VMEM((chunk + _L,), jnp.int32),
            pltpu.VMEM((_L, 2 * 64), jnp.float32),
            pltpu.VMEM((nb, 64), jnp.float32),
            pltpu.VMEM((nb * _L,), jnp.float32),
            pltpu.SemaphoreType.DMA,
        ],
    )
    def ka(xs_hbm, src_hbm, dst_hbm, guards_hbm, ini_hbm, fin_hbm, part_hbm,
           src_v, dst_v, ini_v, fin_v, coefc_v, idxc_v, rows_v,
           xs_v, out_v, sem):
        wid = lax.axis_index("s") * 2 + lax.axis_index("c")
        base = wid * chunk
        lanes = lax.broadcasted_iota(jnp.int32, (_L,), 0)
        pltpu.sync_copy(xs_hbm, xs_v)
        pltpu.sync_copy(ini_hbm, ini_v)
        pltpu.sync_copy(fin_hbm, fin_v)
        pltpu.sync_copy(src_hbm.at[pl.ds(base, chunk)], src_v)
        pltpu.sync_copy(dst_hbm.at[pl.ds(base, chunk)], dst_v)

        zeros_f = jnp.zeros((_L,), jnp.float32)
        zeros_i = jnp.zeros((_L,), jnp.int32)

        def zero_body(j, carry):
            o = j * _L
            coefc_v[pl.ds(o, _L)] = zeros_f
            idxc_v[pl.ds(o, _L)] = zeros_i
            return carry

        lax.fori_loop(0, n_steps + 1, zero_body, 0)

        def scan_body(j, slot):
            o = j * _L
            s_idx = src_v[pl.ds(o, _L)]
            d_idx = dst_v[pl.ds(o, _L)]
            c = plsc.load_gather(ini_v, [s_idx]) * plsc.load_gather(fin_v, [d_idx])
            e_vec = (base + o) + lanes
            m = (c != 0.0) & (e_vec < e_total)
            plsc.store_compressed(coefc_v.at[pl.ds(slot * _L, _L)], c, mask=m)
            plsc.store_compressed(idxc_v.at[pl.ds(slot * _L, _L)], e_vec, mask=m)
            any_set = jnp.sum(m.astype(jnp.int32)) > 0
            return slot + any_set.astype(jnp.int32)

        n_chunks = lax.fori_loop(0, n_steps, scan_body, jnp.int32(0))

        def dot_body(k, parts):
            o = k * _L
            idxv = idxc_v[pl.ds(o, _L)]
            colb = (idxv % 2) * d
            coefv = coefc_v[pl.ds(o, _L)]
            pltpu.async_copy(guards_hbm.at[idxv // 2], rows_v, sem).wait()
            logits = [jnp.zeros((_L,), jnp.float32) for _ in range(nb)]
            for f in range(d):
                tvec = plsc.load_gather(rows_v, [lanes, colb + f])
                for b_ in range(nb):
                    xsp = plsc.load_gather(
                        xs_v, [jnp.full((_L,), b_, jnp.int32),
                               jnp.full((_L,), f, jnp.int32)])
                    logits[b_] = logits[b_] + tvec * xsp
            return tuple(parts[b_] + _sigmoid(logits[b_]) * coefv
                         for b_ in range(nb))

        parts = lax.fori_loop(
            0, n_chunks, dot_body,
            tuple(jnp.zeros((_L,), jnp.float32) for _ in range(nb)))

        for b_ in range(nb):
            out_v[pl.ds(b_ * _L, _L)] = parts[b_]
        pltpu.sync_copy(out_v, part_hbm.at[pl.ds(wid * nb * _L, nb * _L)])

    return ka(xs, src_pad, dst_pad, guards2, ini, fin)


def _stage_b(partials, nb):
    mesh = plsc.VectorSubcoreMesh(core_axis_name="c", subcore_axis_name="s")
    n_part = partials.shape[0]

    @functools.partial(
        pl.kernel,
        out_type=jax.ShapeDtypeStruct((_L,), jnp.float32),
        mesh=mesh,
        compiler_params=pltpu.CompilerParams(needs_layout_passes=False),
        scratch_types=[
            pltpu.VMEM((n_part,), jnp.float32),
            pltpu.VMEM((_L,), jnp.float32),
        ],
    )
    def kb(part_hbm, out_hbm, pa_v, out_v):
        wid = lax.axis_index("s") * 2 + lax.axis_index("c")

        lanes = lax.broadcasted_iota(jnp.int32, (_L,), 0)

        @pl.when(wid == 0)
        def _():
            pltpu.sync_copy(part_hbm, pa_v)
            out_v[...] = jnp.zeros((_L,), jnp.float32)
            for b_ in range(nb):
                def red(t, acc):
                    return acc + pa_v[pl.ds(t * nb * _L + b_ * _L, _L)]
                acc = lax.fori_loop(0, _NW, red, jnp.zeros((_L,), jnp.float32))
                s = jnp.sum(acc)
                plsc.store_scatter(
                    out_v, [jnp.full((_L,), b_, jnp.int32)],
                    jnp.full((_L,), s, jnp.float32), mask=lanes == 0)
            pltpu.sync_copy(out_v, out_hbm)

    return kb(partials)


def kernel(xs, src, dst, guards, initial_weights, final_weights):
    b, d = xs.shape
    e = src.shape[0]
    e_pad = -(-e // (_NW * _L)) * (_NW * _L)
    src_pad = jnp.pad(src, (0, e_pad - e))
    dst_pad = jnp.pad(dst, (0, e_pad - e))
    guards2 = guards.reshape(e // 2, 2 * d)
    partials = _stage_a(xs, src_pad, dst_pad, guards2, e,
                        initial_weights, final_weights)
    out16 = _stage_b(partials, b)
    return out16[:b]


# speedup vs baseline: 1.5677x; 1.5677x over previous
import functools

import jax
import jax.numpy as jnp
from jax import lax
from jax.experimental import pallas as pl
from jax.experimental.pallas import tpu as pltpu
from jax.experimental.pallas import tpu_sc as plsc

_NW = 32
_L = 16


def _sigmoid(x):
    return 1.0 / (1.0 + jnp.exp(-x))


def _stage_a(xs_splat, nb, d, src_pad, dst_pad, guards, e_total, ini, fin):
    e_pad = src_pad.shape[0]
    q = ini.shape[0]
    chunk = e_pad // _NW
    n_steps = chunk // _L
    unroll = 4
    mesh = plsc.VectorSubcoreMesh(core_axis_name="c", subcore_axis_name="s")

    @functools.partial(
        pl.kernel,
        out_type=jax.ShapeDtypeStruct((_NW * nb * _L,), jnp.float32),
        mesh=mesh,
        compiler_params=pltpu.CompilerParams(needs_layout_passes=False),
        scratch_types=[
            pltpu.VMEM((chunk,), jnp.int32),
            pltpu.VMEM((chunk,), jnp.int32),
            pltpu.VMEM((q,), jnp.float32),
            pltpu.VMEM((q,), jnp.float32),
            pltpu.VMEM((chunk + _L,), jnp.float32),
            pltpu.VMEM((chunk + _L,), jnp.int32),
            pltpu.VMEM((_L, 64), jnp.float32),
            pltpu.VMEM((nb * d * _L,), jnp.float32),
            pltpu.VMEM((nb * _L,), jnp.float32),
            pltpu.SemaphoreType.DMA,
        ],
    )
    def ka(xs_hbm, src_hbm, dst_hbm, guards_hbm, ini_hbm, fin_hbm, part_hbm,
           src_v, dst_v, ini_v, fin_v, coefc_v, orig_v, rows_v,
           xs_v, out_v, sem):
        wid = lax.axis_index("s") * 2 + lax.axis_index("c")
        base = wid * chunk
        lanes = lax.broadcasted_iota(jnp.int32, (_L,), 0)
        pltpu.sync_copy(xs_hbm, xs_v)
        pltpu.sync_copy(ini_hbm, ini_v)
        pltpu.sync_copy(fin_hbm, fin_v)
        pltpu.sync_copy(src_hbm.at[pl.ds(base, chunk)], src_v)
        pltpu.sync_copy(dst_hbm.at[pl.ds(base, chunk)], dst_v)

        def scan_step(o, slot):
            s_idx = src_v[pl.ds(o, _L)]
            d_idx = dst_v[pl.ds(o, _L)]
            c = plsc.load_gather(ini_v, [s_idx]) * plsc.load_gather(fin_v, [d_idx])
            e_vec = (base + o) + lanes
            m = (c != 0.0) & (e_vec < e_total)
            coefc_v[pl.ds(slot * _L, _L)] = jnp.where(m, c, 0.0)
            orig_v[pl.ds(slot * _L, _L)] = jnp.full((_L,), base, jnp.int32) + o
            return slot + jnp.any(m).astype(jnp.int32)

        def scan_body(j, slot):
            o = j * (_L * unroll)
            for u in range(unroll):
                slot = scan_step(o + u * _L, slot)
            return slot

        n_chunks = lax.fori_loop(0, n_steps // unroll, scan_body, jnp.int32(0))

        def dot_body(k, parts):
            o = k * _L
            eo = pl.multiple_of(jnp.max(orig_v[pl.ds(o, _L)]), _L)
            coefv = coefc_v[pl.ds(o, _L)]
            pltpu.sync_copy(guards_hbm.at[pl.ds(eo, _L)], rows_v)
            logits = [jnp.zeros((_L,), jnp.float32) for _ in range(nb)]
            for f in range(d):
                tvec = plsc.load_gather(rows_v, [lanes, jnp.full((_L,), f, jnp.int32)])
                for b_ in range(nb):
                    xsp = xs_v[pl.ds((b_ * d + f) * _L, _L)]
                    logits[b_] = logits[b_] + tvec * xsp
            return tuple(parts[b_] + jnp.where(coefv != 0.0,
                                               _sigmoid(logits[b_]) * coefv,
                                               0.0)
                         for b_ in range(nb))

        parts = lax.fori_loop(
            0, n_chunks, dot_body,
            tuple(jnp.zeros((_L,), jnp.float32) for _ in range(nb)))

        for b_ in range(nb):
            out_v[pl.ds(b_ * _L, _L)] = parts[b_]
        pltpu.sync_copy(out_v, part_hbm.at[pl.ds(wid * nb * _L, nb * _L)])

    return ka(xs_splat, src_pad, dst_pad, guards, ini, fin)


def _stage_b(partials, nb):
    mesh = plsc.VectorSubcoreMesh(core_axis_name="c", subcore_axis_name="s")
    n_part = partials.shape[0]

    @functools.partial(
        pl.kernel,
        out_type=jax.ShapeDtypeStruct((_L,), jnp.float32),
        mesh=mesh,
        compiler_params=pltpu.CompilerParams(needs_layout_passes=False),
        scratch_types=[
            pltpu.VMEM((n_part,), jnp.float32),
            pltpu.VMEM((_L,), jnp.float32),
        ],
    )
    def kb(part_hbm, out_hbm, pa_v, out_v):
        wid = lax.axis_index("s") * 2 + lax.axis_index("c")

        lanes = lax.broadcasted_iota(jnp.int32, (_L,), 0)

        @pl.when(wid == 0)
        def _():
            pltpu.sync_copy(part_hbm, pa_v)
            out_v[...] = jnp.zeros((_L,), jnp.float32)
            for b_ in range(nb):
                def red(t, acc):
                    return acc + pa_v[pl.ds(t * nb * _L + b_ * _L, _L)]
                acc = lax.fori_loop(0, _NW, red, jnp.zeros((_L,), jnp.float32))
                s = jnp.sum(acc)
                plsc.store_scatter(
                    out_v, [jnp.full((_L,), b_, jnp.int32)],
                    jnp.full((_L,), s, jnp.float32), mask=lanes == 0)
            pltpu.sync_copy(out_v, out_hbm)

    return kb(partials)


def kernel(xs, src, dst, guards, initial_weights, final_weights):
    b, d = xs.shape
    e = src.shape[0]
    e_pad = -(-e // (_NW * _L)) * (_NW * _L)
    src_pad = jnp.pad(src, (0, e_pad - e))
    dst_pad = jnp.pad(dst, (0, e_pad - e))
    xs_splat = jnp.broadcast_to(xs[:, :, None], (b, d, _L)).reshape(-1)
    partials = _stage_a(xs_splat, b, d, src_pad, dst_pad, guards, e,
                        initial_weights, final_weights)
    out16 = _stage_b(partials, b)
    return out16[:b]


# speedup vs baseline: 1.6027x; 1.0223x over previous
import functools

import jax
import jax.numpy as jnp
from jax import lax
from jax.experimental import pallas as pl
from jax.experimental.pallas import tpu as pltpu
from jax.experimental.pallas import tpu_sc as plsc

_NW = 32
_L = 16


def _sigmoid(x):
    return 1.0 / (1.0 + jnp.exp(-x))


def _stage_a(xs_splat, nb, d, src_pad, dst_pad, guards, e_total, ini, fin):
    e_pad = src_pad.shape[0]
    q = ini.shape[0]
    chunk = e_pad // _NW
    n_steps = chunk // _L
    unroll = 4
    mesh = plsc.VectorSubcoreMesh(core_axis_name="c", subcore_axis_name="s")

    @functools.partial(
        pl.kernel,
        out_type=jax.ShapeDtypeStruct((_NW * nb * _L,), jnp.float32),
        mesh=mesh,
        compiler_params=pltpu.CompilerParams(needs_layout_passes=False),
        scratch_types=[
            pltpu.VMEM((chunk,), jnp.int32),
            pltpu.VMEM((chunk,), jnp.int32),
            pltpu.VMEM((q,), jnp.float32),
            pltpu.VMEM((q,), jnp.float32),
            pltpu.VMEM((chunk + _L,), jnp.float32),
            pltpu.VMEM((chunk + _L,), jnp.int32),
            pltpu.VMEM((_L, 64), jnp.float32),
            pltpu.VMEM((nb * d * _L,), jnp.float32),
            pltpu.VMEM((nb * _L,), jnp.float32),
            pltpu.SemaphoreType.DMA,
        ],
    )
    def ka(xs_hbm, src_hbm, dst_hbm, guards_hbm, ini_hbm, fin_hbm, part_hbm,
           src_v, dst_v, ini_v, fin_v, coefc_v, orig_v, rows_v,
           xs_v, out_v, sem):
        wid = lax.axis_index("s") * 2 + lax.axis_index("c")
        base = wid * chunk
        lanes = lax.broadcasted_iota(jnp.int32, (_L,), 0)
        pltpu.sync_copy(xs_hbm, xs_v)
        pltpu.sync_copy(ini_hbm, ini_v)
        pltpu.sync_copy(fin_hbm, fin_v)
        pltpu.sync_copy(src_hbm.at[pl.ds(base, chunk)], src_v)
        pltpu.sync_copy(dst_hbm.at[pl.ds(base, chunk)], dst_v)

        def scan_step(o, slot_vec):
            s_idx = src_v[pl.ds(o, _L)]
            d_idx = dst_v[pl.ds(o, _L)]
            c = plsc.load_gather(ini_v, [s_idx]) * plsc.load_gather(fin_v, [d_idx])
            e_vec = (base + o) + lanes
            m = (c != 0.0) & (e_vec < e_total)
            pos = slot_vec * _L + lanes
            plsc.store_scatter(coefc_v, [pos], jnp.where(m, c, 0.0))
            plsc.store_scatter(orig_v, [pos], jnp.full((_L,), base, jnp.int32) + o)
            pc = plsc.all_reduce_population_count(m)
            return slot_vec + (pc > 0).astype(jnp.int32)

        def scan_body(j, slot_vec):
            o = j * (_L * unroll)
            for u in range(unroll):
                slot_vec = scan_step(o + u * _L, slot_vec)
            return slot_vec

        slot_vec = lax.fori_loop(0, n_steps // unroll, scan_body,
                                 jnp.zeros((_L,), jnp.int32))
        n_chunks = jnp.max(slot_vec)

        def dot_body(k, parts):
            o = k * _L
            eo = pl.multiple_of(jnp.max(orig_v[pl.ds(o, _L)]), _L)
            coefv = coefc_v[pl.ds(o, _L)]
            pltpu.sync_copy(guards_hbm.at[pl.ds(eo, _L)], rows_v)
            logits = [jnp.zeros((_L,), jnp.float32) for _ in range(nb)]
            for f in range(d):
                tvec = plsc.load_gather(rows_v, [lanes, jnp.full((_L,), f, jnp.int32)])
                for b_ in range(nb):
                    xsp = xs_v[pl.ds((b_ * d + f) * _L, _L)]
                    logits[b_] = logits[b_] + tvec * xsp
            return tuple(parts[b_] + jnp.where(coefv != 0.0,
                                               _sigmoid(logits[b_]) * coefv,
                                               0.0)
                         for b_ in range(nb))

        parts = lax.fori_loop(
            0, n_chunks, dot_body,
            tuple(jnp.zeros((_L,), jnp.float32) for _ in range(nb)))

        for b_ in range(nb):
            out_v[pl.ds(b_ * _L, _L)] = parts[b_]
        pltpu.sync_copy(out_v, part_hbm.at[pl.ds(wid * nb * _L, nb * _L)])

    return ka(xs_splat, src_pad, dst_pad, guards, ini, fin)


def _stage_b(partials, nb):
    mesh = plsc.VectorSubcoreMesh(core_axis_name="c", subcore_axis_name="s")
    n_part = partials.shape[0]

    @functools.partial(
        pl.kernel,
        out_type=jax.ShapeDtypeStruct((_L,), jnp.float32),
        mesh=mesh,
        compiler_params=pltpu.CompilerParams(needs_layout_passes=False),
        scratch_types=[
            pltpu.VMEM((n_part,), jnp.float32),
            pltpu.VMEM((_L,), jnp.float32),
        ],
    )
    def kb(part_hbm, out_hbm, pa_v, out_v):
        wid = lax.axis_index("s") * 2 + lax.axis_index("c")

        lanes = lax.broadcasted_iota(jnp.int32, (_L,), 0)

        @pl.when(wid == 0)
        def _():
            pltpu.sync_copy(part_hbm, pa_v)
            out_v[...] = jnp.zeros((_L,), jnp.float32)
            for b_ in range(nb):
                def red(t, acc):
                    return acc + pa_v[pl.ds(t * nb * _L + b_ * _L, _L)]
                acc = lax.fori_loop(0, _NW, red, jnp.zeros((_L,), jnp.float32))
                s = jnp.sum(acc)
                plsc.store_scatter(
                    out_v, [jnp.full((_L,), b_, jnp.int32)],
                    jnp.full((_L,), s, jnp.float32), mask=lanes == 0)
            pltpu.sync_copy(out_v, out_hbm)

    return kb(partials)


def kernel(xs, src, dst, guards, initial_weights, final_weights):
    b, d = xs.shape
    e = src.shape[0]
    e_pad = -(-e // (_NW * _L)) * (_NW * _L)
    src_pad = jnp.pad(src, (0, e_pad - e))
    dst_pad = jnp.pad(dst, (0, e_pad - e))
    xs_splat = jnp.broadcast_to(xs[:, :, None], (b, d, _L)).reshape(-1)
    partials = _stage_a(xs_splat, b, d, src_pad, dst_pad, guards, e,
                        initial_weights, final_weights)
    out16 = _stage_b(partials, b)
    return out16[:b]


# speedup vs baseline: 2.6950x; 1.6816x over previous
import functools

import jax
import jax.numpy as jnp
from jax import lax
from jax.experimental import pallas as pl
from jax.experimental.pallas import tpu as pltpu
from jax.experimental.pallas import tpu_sc as plsc

_NW = 32
_L = 16
_W = 128


def _sigmoid(x):
    return 1.0 / (1.0 + jnp.exp(-x))


def _stage_a(xs_splat, nb, d, src_pad, dst_pad, guards_t, e_total, ini, fin):
    e_pad = src_pad.shape[0]
    q = ini.shape[0]
    chunk = e_pad // _NW
    n_win = chunk // _W
    mesh = plsc.VectorSubcoreMesh(core_axis_name="c", subcore_axis_name="s")

    @functools.partial(
        pl.kernel,
        out_type=jax.ShapeDtypeStruct((_NW * nb * _L,), jnp.float32),
        mesh=mesh,
        compiler_params=pltpu.CompilerParams(needs_layout_passes=False),
        scratch_types=[
            pltpu.VMEM((chunk,), jnp.int32),
            pltpu.VMEM((chunk,), jnp.int32),
            pltpu.VMEM((q,), jnp.float32),
            pltpu.VMEM((q,), jnp.float32),
            pltpu.VMEM((chunk + _W,), jnp.float32),
            pltpu.VMEM(((n_win + 1) * _L,), jnp.int32),
            pltpu.VMEM((d, _W), jnp.float32),
            pltpu.VMEM((nb * d * _L,), jnp.float32),
            pltpu.VMEM((nb * _L,), jnp.float32),
            pltpu.SemaphoreType.DMA,
        ],
    )
    def ka(xs_hbm, src_hbm, dst_hbm, guards_hbm, ini_hbm, fin_hbm, part_hbm,
           src_v, dst_v, ini_v, fin_v, coefc_v, orig_v, rows_v,
           xs_v, out_v, sem):
        wid = lax.axis_index("s") * 2 + lax.axis_index("c")
        base = wid * chunk
        lanes = lax.broadcasted_iota(jnp.int32, (_L,), 0)
        pltpu.sync_copy(xs_hbm, xs_v)
        pltpu.sync_copy(ini_hbm, ini_v)
        pltpu.sync_copy(fin_hbm, fin_v)
        pltpu.sync_copy(src_hbm.at[pl.ds(base, chunk)], src_v)
        pltpu.sync_copy(dst_hbm.at[pl.ds(base, chunk)], dst_v)

        def scan_win(w, slot_vec):
            o = w * _W
            any_mask = jnp.zeros((_L,), jnp.bool_)
            for u in range(_W // _L):
                ou = o + u * _L
                s_idx = src_v[pl.ds(ou, _L)]
                d_idx = dst_v[pl.ds(ou, _L)]
                c = (plsc.load_gather(ini_v, [s_idx])
                     * plsc.load_gather(fin_v, [d_idx]))
                e_vec = (base + ou) + lanes
                m = (c != 0.0) & (e_vec < e_total)
                plsc.store_scatter(coefc_v, [slot_vec * _W + (u * _L) + lanes],
                                   jnp.where(m, c, 0.0))
                any_mask = any_mask | m
            plsc.store_scatter(orig_v, [slot_vec * _L + lanes],
                               jnp.full((_L,), base, jnp.int32) + o)
            pc = plsc.all_reduce_population_count(any_mask)
            return slot_vec + (pc > 0).astype(jnp.int32)

        slot_vec = lax.fori_loop(0, n_win, scan_win,
                                 jnp.zeros((_L,), jnp.int32))
        n_slots = jnp.max(slot_vec)

        def dot_body(k, parts):
            eo = pl.multiple_of(jnp.max(orig_v[pl.ds(k * _L, _L)]), _W)
            pltpu.sync_copy(guards_hbm.at[:, pl.ds(eo, _W)], rows_v)
            for g in range(_W // _L):
                coefv = coefc_v[pl.ds(k * _W + g * _L, _L)]
                logits = [jnp.zeros((_L,), jnp.float32) for _ in range(nb)]
                for f in range(d):
                    val = rows_v[f, pl.ds(g * _L, _L)]
                    for b_ in range(nb):
                        xsp = xs_v[pl.ds((b_ * d + f) * _L, _L)]
                        logits[b_] = logits[b_] + val * xsp
                parts = tuple(
                    parts[b_] + jnp.where(coefv != 0.0,
                                          _sigmoid(logits[b_]) * coefv, 0.0)
                    for b_ in range(nb))
            return parts

        parts = lax.fori_loop(
            0, n_slots, dot_body,
            tuple(jnp.zeros((_L,), jnp.float32) for _ in range(nb)))

        for b_ in range(nb):
            out_v[pl.ds(b_ * _L, _L)] = parts[b_]
        pltpu.sync_copy(out_v, part_hbm.at[pl.ds(wid * nb * _L, nb * _L)])

    return ka(xs_splat, src_pad, dst_pad, guards_t, ini, fin)


def _stage_b(partials, nb):
    mesh = plsc.VectorSubcoreMesh(core_axis_name="c", subcore_axis_name="s")
    n_part = partials.shape[0]

    @functools.partial(
        pl.kernel,
        out_type=jax.ShapeDtypeStruct((_L,), jnp.float32),
        mesh=mesh,
        compiler_params=pltpu.CompilerParams(needs_layout_passes=False),
        scratch_types=[
            pltpu.VMEM((n_part,), jnp.float32),
            pltpu.VMEM((_L,), jnp.float32),
        ],
    )
    def kb(part_hbm, out_hbm, pa_v, out_v):
        wid = lax.axis_index("s") * 2 + lax.axis_index("c")
        lanes = lax.broadcasted_iota(jnp.int32, (_L,), 0)

        @pl.when(wid == 0)
        def _():
            pltpu.sync_copy(part_hbm, pa_v)
            out_v[...] = jnp.zeros((_L,), jnp.float32)
            for b_ in range(nb):
                def red(t, acc):
                    return acc + pa_v[pl.ds(t * nb * _L + b_ * _L, _L)]
                acc = lax.fori_loop(0, _NW, red, jnp.zeros((_L,), jnp.float32))
                s = jnp.sum(acc)
                plsc.store_scatter(
                    out_v, [jnp.full((_L,), b_, jnp.int32)],
                    jnp.full((_L,), s, jnp.float32), mask=lanes == 0)
            pltpu.sync_copy(out_v, out_hbm)

    return kb(partials)


def kernel(xs, src, dst, guards, initial_weights, final_weights):
    b, d = xs.shape
    e = src.shape[0]
    e_pad = -(-e // (_NW * _W)) * (_NW * _W)
    src_pad = jnp.pad(src, (0, e_pad - e))
    dst_pad = jnp.pad(dst, (0, e_pad - e))
    guards_t = guards.T
    xs_splat = jnp.broadcast_to(xs[:, :, None], (b, d, _L)).reshape(-1)
    partials = _stage_a(xs_splat, b, d, src_pad, dst_pad, guards_t, e,
                        initial_weights, final_weights)
    out16 = _stage_b(partials, b)
    return out16[:b]


# speedup vs baseline: 3.7988x; 1.4096x over previous
import functools

import jax
import jax.numpy as jnp
from jax import lax
from jax.experimental import pallas as pl
from jax.experimental.pallas import tpu as pltpu
from jax.experimental.pallas import tpu_sc as plsc

_NW = 32
_L = 16
_W = 128


def _sigmoid(x):
    return 1.0 / (1.0 + jnp.exp(-x))


def _stage_a(xs_splat, nb, d, src_pad, dst_pad, guards_t, e_total, ini, fin):
    e_pad = src_pad.shape[0]
    q = ini.shape[0]
    chunk = e_pad // _NW
    n_win = chunk // _W
    mesh = plsc.VectorSubcoreMesh(core_axis_name="c", subcore_axis_name="s")

    @functools.partial(
        pl.kernel,
        out_type=jax.ShapeDtypeStruct((_NW * nb * _L,), jnp.float32),
        mesh=mesh,
        compiler_params=pltpu.CompilerParams(needs_layout_passes=False),
        scratch_types=[
            pltpu.VMEM((chunk,), jnp.int32),
            pltpu.VMEM((chunk,), jnp.int32),
            pltpu.VMEM((q,), jnp.float32),
            pltpu.VMEM((q,), jnp.float32),
            pltpu.VMEM((chunk + _W,), jnp.float32),
            pltpu.VMEM(((n_win + 1) * _L,), jnp.int32),
            pltpu.VMEM((d, _W), jnp.float32),
            pltpu.VMEM((nb * d * _L,), jnp.float32),
            pltpu.VMEM((nb * _L,), jnp.float32),
            pltpu.SemaphoreType.DMA,
        ],
    )
    def ka(xs_hbm, src_hbm, dst_hbm, guards_hbm, ini_hbm, fin_hbm, part_hbm,
           src_v, dst_v, ini_v, fin_v, coefc_v, orig_v, rows_v,
           xs_v, out_v, sem):
        wid = lax.axis_index("s") * 2 + lax.axis_index("c")
        base = wid * chunk
        lanes = lax.broadcasted_iota(jnp.int32, (_L,), 0)
        pltpu.sync_copy(xs_hbm, xs_v)
        pltpu.sync_copy(ini_hbm, ini_v)
        pltpu.sync_copy(fin_hbm, fin_v)
        pltpu.sync_copy(src_hbm.at[pl.ds(base, chunk)], src_v)
        pltpu.sync_copy(dst_hbm.at[pl.ds(base, chunk)], dst_v)

        def scan_win(w, slot_vec):
            o = w * _W
            any_mask = jnp.zeros((_L,), jnp.bool_)
            for u in range(_W // _L):
                ou = o + u * _L
                s_idx = src_v[pl.ds(ou, _L)]
                d_idx = dst_v[pl.ds(ou, _L)]
                c = (plsc.load_gather(ini_v, [s_idx])
                     * plsc.load_gather(fin_v, [d_idx]))
                e_vec = (base + ou) + lanes
                m = (c != 0.0) & (e_vec < e_total)
                plsc.store_scatter(coefc_v, [slot_vec * _W + (u * _L) + lanes],
                                   jnp.where(m, c, 0.0))
                any_mask = any_mask | m
            plsc.store_scatter(orig_v, [slot_vec * _L + lanes],
                               jnp.full((_L,), base, jnp.int32) + o)
            pc = plsc.all_reduce_population_count(any_mask)
            return slot_vec + (pc > 0).astype(jnp.int32)

        slot_vec = lax.fori_loop(0, n_win, scan_win,
                                 jnp.zeros((_L,), jnp.int32))
        n_slots = jnp.max(slot_vec)

        def dot_body(k, parts):
            eo = pl.multiple_of(jnp.max(orig_v[pl.ds(k * _L, _L)]), _W)
            pltpu.sync_copy(guards_hbm.at[:, pl.ds(eo, _W)], rows_v)
            for g in range(_W // _L):
                coefv = coefc_v[pl.ds(k * _W + g * _L, _L)]

                def do_group(p, g=g, coefv=coefv):
                    def fdot(f, logits):
                        val = rows_v[f, pl.ds(g * _L, _L)]
                        return tuple(
                            logits[b_] + val * xs_v[pl.ds(b_ * d * _L + f * _L, _L)]
                            for b_ in range(nb))
                    logits = lax.fori_loop(
                        0, d, fdot,
                        tuple(jnp.zeros((_L,), jnp.float32) for _ in range(nb)))
                    return tuple(
                        p[b_] + jnp.where(coefv != 0.0,
                                          _sigmoid(logits[b_]) * coefv, 0.0)
                        for b_ in range(nb))

                parts = lax.cond(jnp.any(coefv != 0.0), do_group,
                                 lambda p: p, parts)
            return parts

        parts = lax.fori_loop(
            0, n_slots, dot_body,
            tuple(jnp.zeros((_L,), jnp.float32) for _ in range(nb)))

        for b_ in range(nb):
            out_v[pl.ds(b_ * _L, _L)] = parts[b_]
        pltpu.sync_copy(out_v, part_hbm.at[pl.ds(wid * nb * _L, nb * _L)])

    return ka(xs_splat, src_pad, dst_pad, guards_t, ini, fin)


def _stage_b(partials, nb):
    mesh = plsc.VectorSubcoreMesh(core_axis_name="c", subcore_axis_name="s")
    n_part = partials.shape[0]

    @functools.partial(
        pl.kernel,
        out_type=jax.ShapeDtypeStruct((_L,), jnp.float32),
        mesh=mesh,
        compiler_params=pltpu.CompilerParams(needs_layout_passes=False),
        scratch_types=[
            pltpu.VMEM((n_part,), jnp.float32),
            pltpu.VMEM((_L,), jnp.float32),
        ],
    )
    def kb(part_hbm, out_hbm, pa_v, out_v):
        wid = lax.axis_index("s") * 2 + lax.axis_index("c")
        lanes = lax.broadcasted_iota(jnp.int32, (_L,), 0)

        @pl.when(wid == 0)
        def _():
            pltpu.sync_copy(part_hbm, pa_v)
            out_v[...] = jnp.zeros((_L,), jnp.float32)
            for b_ in range(nb):
                def red(t, acc):
                    return acc + pa_v[pl.ds(t * nb * _L + b_ * _L, _L)]
                acc = lax.fori_loop(0, _NW, red, jnp.zeros((_L,), jnp.float32))
                s = jnp.sum(acc)
                plsc.store_scatter(
                    out_v, [jnp.full((_L,), b_, jnp.int32)],
                    jnp.full((_L,), s, jnp.float32), mask=lanes == 0)
            pltpu.sync_copy(out_v, out_hbm)

    return kb(partials)


def kernel(xs, src, dst, guards, initial_weights, final_weights):
    b, d = xs.shape
    e = src.shape[0]
    e_pad = -(-e // (_NW * _W)) * (_NW * _W)
    src_pad = jnp.pad(src, (0, e_pad - e))
    dst_pad = jnp.pad(dst, (0, e_pad - e))
    guards_t = guards.T
    xs_splat = jnp.broadcast_to(xs[:, :, None], (b, d, _L)).reshape(-1)
    partials = _stage_a(xs_splat, b, d, src_pad, dst_pad, guards_t, e,
                        initial_weights, final_weights)
    out16 = _stage_b(partials, b)
    return out16[:b]


# speedup vs baseline: 4.6038x; 1.2119x over previous
import functools

import jax
import jax.numpy as jnp
from jax import lax
from jax.experimental import pallas as pl
from jax.experimental.pallas import tpu as pltpu
from jax.experimental.pallas import tpu_sc as plsc

_NW = 32
_L = 16
_W = 128


def _sigmoid(x):
    return 1.0 / (1.0 + jnp.exp(-x))


def _stage_a(xs_splat, nb, d, e_pad, src, dst, guards_t, e_total, ini, fin):
    q = ini.shape[0]
    chunk = e_pad // _NW
    n_win = chunk // _W
    mesh = plsc.VectorSubcoreMesh(core_axis_name="c", subcore_axis_name="s")

    @functools.partial(
        pl.kernel,
        out_type=jax.ShapeDtypeStruct((_NW * nb * _L,), jnp.float32),
        mesh=mesh,
        compiler_params=pltpu.CompilerParams(needs_layout_passes=False),
        scratch_types=[
            pltpu.VMEM((chunk,), jnp.int32),
            pltpu.VMEM((chunk,), jnp.int32),
            pltpu.VMEM((q,), jnp.float32),
            pltpu.VMEM((q,), jnp.float32),
            pltpu.VMEM((chunk + _W,), jnp.float32),
            pltpu.VMEM(((n_win + 1) * _L,), jnp.int32),
            pltpu.VMEM((d, _W), jnp.float32),
            pltpu.VMEM((nb * d * _L,), jnp.float32),
            pltpu.VMEM((nb * _L,), jnp.float32),
            pltpu.SemaphoreType.DMA,
        ],
    )
    def ka(xs_hbm, src_hbm, dst_hbm, guards_hbm, ini_hbm, fin_hbm, part_hbm,
           src_v, dst_v, ini_v, fin_v, coefc_v, orig_v, rows_v,
           xs_v, out_v, sem):
        wid = lax.axis_index("s") * 2 + lax.axis_index("c")
        base = wid * chunk
        lanes = lax.broadcasted_iota(jnp.int32, (_L,), 0)
        copies = [
            pltpu.async_copy(xs_hbm, xs_v, sem),
            pltpu.async_copy(ini_hbm, ini_v, sem),
            pltpu.async_copy(fin_hbm, fin_v, sem),
            pltpu.async_copy(src_hbm.at[pl.ds(base, chunk)], src_v, sem),
            pltpu.async_copy(dst_hbm.at[pl.ds(base, chunk)], dst_v, sem),
        ]
        for cp in copies:
            cp.wait()

        def scan_win(w, slot_vec):
            o = w * _W
            any_mask = jnp.zeros((_L,), jnp.bool_)
            gathered = []
            for u in range(_W // _L):
                ou = o + u * _L
                s_idx = jnp.clip(src_v[pl.ds(ou, _L)], 0, q - 1)
                d_idx = jnp.clip(dst_v[pl.ds(ou, _L)], 0, q - 1)
                gathered.append((plsc.load_gather(ini_v, [s_idx]),
                                 plsc.load_gather(fin_v, [d_idx])))
            for u in range(_W // _L):
                a, bb = gathered[u]
                c = a * bb
                e_vec = (base + (o + u * _L)) + lanes
                m = (c != 0.0) & (e_vec < e_total)
                plsc.store_scatter(coefc_v, [slot_vec * _W + (u * _L) + lanes],
                                   jnp.where(m, c, 0.0))
                any_mask = any_mask | m
            plsc.store_scatter(orig_v, [slot_vec * _L + lanes],
                               jnp.full((_L,), base, jnp.int32) + o)
            pc = plsc.all_reduce_population_count(any_mask)
            return slot_vec + (pc > 0).astype(jnp.int32)

        slot_vec = lax.fori_loop(0, n_win, scan_win,
                                 jnp.zeros((_L,), jnp.int32))
        n_slots = jnp.max(slot_vec)

        def dot_body(k, parts):
            eo = pl.multiple_of(jnp.max(orig_v[pl.ds(k * _L, _L)]), _W)
            pltpu.sync_copy(guards_hbm.at[:, pl.ds(eo, _W)], rows_v)
            for g in range(_W // _L):
                coefv = coefc_v[pl.ds(k * _W + g * _L, _L)]

                def do_group(p, g=g, coefv=coefv):
                    def fdot(f, logits):
                        val = rows_v[f, pl.ds(g * _L, _L)]
                        return tuple(
                            logits[b_] + val * xs_v[pl.ds(b_ * d * _L + f * _L, _L)]
                            for b_ in range(nb))
                    logits = lax.fori_loop(
                        0, d, fdot,
                        tuple(jnp.zeros((_L,), jnp.float32) for _ in range(nb)))
                    return tuple(
                        p[b_] + jnp.where(coefv != 0.0,
                                          _sigmoid(logits[b_]) * coefv, 0.0)
                        for b_ in range(nb))

                parts = lax.cond(jnp.any(coefv != 0.0), do_group,
                                 lambda p: p, parts)
            return parts

        parts = lax.fori_loop(
            0, n_slots, dot_body,
            tuple(jnp.zeros((_L,), jnp.float32) for _ in range(nb)))

        for b_ in range(nb):
            out_v[pl.ds(b_ * _L, _L)] = parts[b_]
        pltpu.sync_copy(out_v, part_hbm.at[pl.ds(wid * nb * _L, nb * _L)])

    return ka(xs_splat, src, dst, guards_t, ini, fin)


def _stage_b(partials, nb):
    mesh = plsc.VectorSubcoreMesh(core_axis_name="c", subcore_axis_name="s")
    n_part = partials.shape[0]

    @functools.partial(
        pl.kernel,
        out_type=jax.ShapeDtypeStruct((_L,), jnp.float32),
        mesh=mesh,
        compiler_params=pltpu.CompilerParams(needs_layout_passes=False),
        scratch_types=[
            pltpu.VMEM((n_part,), jnp.float32),
            pltpu.VMEM((_L,), jnp.float32),
        ],
    )
    def kb(part_hbm, out_hbm, pa_v, out_v):
        wid = lax.axis_index("s") * 2 + lax.axis_index("c")
        lanes = lax.broadcasted_iota(jnp.int32, (_L,), 0)

        @pl.when(wid == 0)
        def _():
            pltpu.sync_copy(part_hbm, pa_v)
            out_v[...] = jnp.zeros((_L,), jnp.float32)
            for b_ in range(nb):
                def red(t, acc):
                    return acc + pa_v[pl.ds(t * nb * _L + b_ * _L, _L)]
                acc = lax.fori_loop(0, _NW, red, jnp.zeros((_L,), jnp.float32))
                s = jnp.sum(acc)
                plsc.store_scatter(
                    out_v, [jnp.full((_L,), b_, jnp.int32)],
                    jnp.full((_L,), s, jnp.float32), mask=lanes == 0)
            pltpu.sync_copy(out_v, out_hbm)

    return kb(partials)


def kernel(xs, src, dst, guards, initial_weights, final_weights):
    b, d = xs.shape
    e = src.shape[0]
    e_pad = -(-e // (_NW * _W)) * (_NW * _W)
    guards_t = guards.T
    xs_splat = jnp.broadcast_to(xs[:, :, None], (b, d, _L)).reshape(-1)
    partials = _stage_a(xs_splat, b, d, e_pad, src, dst, guards_t, e,
                        initial_weights, final_weights)
    out16 = _stage_b(partials, b)
    return out16[:b]
